# baseline (speedup 1.0000x reference)
import functools

import jax
import jax.numpy as jnp
from jax import lax
from jax.experimental import pallas as pl
from jax.experimental.pallas import tpu as pltpu, tpu_sc as plsc

N = 10000
E = 160000
D = 256
NC = 2
NS = 16
NW = NC * NS
EP = 163840
CH = 128
EW = EP // NW
EW3 = EP // NS
SEC = 2560
HALF = 5120
SPAD = 10752
PAD_DST = 10240


def _tc_dense(x, W_o, b_o, W_u, b_u, pref):
    blk = 1000

    def body(x_ref, wo_ref, bo_ref, wu_ref, bu_ref, pref_ref, hs_ref, p_ref):
        hs_ref[...] = (
            jnp.dot(x_ref[...], wo_ref[...], preferred_element_type=jnp.float32)
            + bo_ref[...]
        )
        p_ref[...] = (
            jnp.dot(pref_ref[...], wu_ref[...], preferred_element_type=jnp.float32)
            + bu_ref[...]
        )

    return pl.pallas_call(
        body,
        grid=(N // blk,),
        in_specs=[
            pl.BlockSpec((blk, D), lambda i: (i, 0)),
            pl.BlockSpec((D, D), lambda i: (0, 0)),
            pl.BlockSpec((1, D), lambda i: (0, 0)),
            pl.BlockSpec((D, D), lambda i: (0, 0)),
            pl.BlockSpec((1, D), lambda i: (0, 0)),
            pl.BlockSpec((blk, D), lambda i: (i, 0)),
        ],
        out_specs=[
            pl.BlockSpec((blk, D), lambda i: (i, 0)),
            pl.BlockSpec((blk, D), lambda i: (i, 0)),
        ],
        out_shape=[
            jax.ShapeDtypeStruct((N, D), jnp.float32),
            jax.ShapeDtypeStruct((N, D), jnp.float32),
        ],
    )(x, W_o, b_o[None, :], W_u, b_u[None, :], pref)


def _sc_logits(hs, p, src, npid, dst, attw, attb):
    mesh = plsc.VectorSubcoreMesh(core_axis_name="c", subcore_axis_name="s")

    @functools.partial(
        pl.kernel,
        mesh=mesh,
        compiler_params=pltpu.CompilerParams(use_tc_tiling_on_sc=False, needs_layout_passes=False),
        out_type=[
            jax.ShapeDtypeStruct((EP,), jnp.float32),
            jax.ShapeDtypeStruct((NC * SPAD,), jnp.float32),
        ],
        scratch_types=[
            pltpu.VMEM((CH,), jnp.int32),
            pltpu.VMEM((CH,), jnp.int32),
            pltpu.VMEM((CH,), jnp.int32),
            pltpu.VMEM((CH, D), jnp.float32),
            pltpu.VMEM((CH, D), jnp.float32),
            pltpu.VMEM((EW,), jnp.float32),
            pltpu.VMEM((D,), jnp.float32),
            pltpu.VMEM((16,), jnp.float32),
            pltpu.VMEM((SPAD // NS,), jnp.float32),
            pltpu.VMEM_SHARED((SPAD,), jnp.float32),
            pltpu.SemaphoreType.DMA,
            pltpu.SemaphoreType.DMA,
            pltpu.SemaphoreType.DMA,
        ],
    )
    def k(hs_hbm, p_hbm, src_hbm, npid_hbm, dst_hbm, attw_hbm, attb_hbm,
          w_out, spart_out, gx, gp, didx,
          rows_a, rows_b, w_all, attw_v, attb_v, zbuf, s_acc,
          sem_a, sem_b, sem_c):
        core = lax.axis_index("c")
        sid = lax.axis_index("s")
        wid = sid * NC + core
        zslice = SPAD // NS

        z16 = jnp.zeros((16,), jnp.float32)

        def zfill(i, _):
            zbuf[pl.ds(i * 16, 16)] = z16
            return 0

        lax.fori_loop(0, zslice // 16, zfill, 0)
        pltpu.sync_copy(zbuf, s_acc.at[pl.ds(sid * zslice, zslice)])

        ebase = wid * EW
        pltpu.sync_copy(attw_hbm, attw_v)
        pltpu.sync_copy(attb_hbm, attb_v)
        plsc.subcore_barrier()

        acc0 = attb_v[...]
        eids = [jnp.arange(16, dtype=jnp.int32) + g * 16 for g in range(CH // 16)]

        def chunk(c, _):
            coff = c * CH
            off = ebase + coff
            i1 = pltpu.async_copy(src_hbm.at[pl.ds(off, CH)], gx, sem_a)
            i2 = pltpu.async_copy(npid_hbm.at[pl.ds(off, CH)], gp, sem_b)
            i3 = pltpu.async_copy(dst_hbm.at[pl.ds(off, CH)], didx, sem_c)
            i1.wait()
            i2.wait()
            i3.wait()
            ca = pltpu.async_copy(hs_hbm.at[gx], rows_a, sem_a)
            cb = pltpu.async_copy(p_hbm.at[gp], rows_b, sem_b)
            ca.wait()
            cb.wait()
            for g in range(CH // 16):
                eg = eids[g]

                def feat(jb, acc):
                    for u in range(4):
                        j = jb * 4 + u
                        js = jnp.full((16,), j, jnp.int32)
                        hv = plsc.load_gather(rows_a, [eg, js])
                        pv = plsc.load_gather(rows_b, [eg, js])
                        aw = plsc.load_gather(attw_v, [js])
                        h = hv + pv
                        lr = jnp.where(h > 0.0, h, h * 0.01)
                        acc = acc + lr * aw
                    return acc

                acc = lax.fori_loop(0, D // 4, feat, acc0)
                w_all[pl.ds(coff + g * 16, 16)] = jnp.exp(acc)
            pltpu.sync_copy(w_all.at[pl.ds(coff, CH)],
                            s_acc.at[didx], add=True)
            return 0

        lax.fori_loop(0, EW // CH, chunk, 0)

        pltpu.sync_copy(w_all, w_out.at[pl.ds(ebase, EW)])
        plsc.subcore_barrier()
        pltpu.sync_copy(s_acc.at[pl.ds(sid * zslice, zslice)],
                        spart_out.at[pl.ds(core * SPAD + sid * zslice, zslice)])

    return k(hs, p, src, npid, dst, attw, attb)


def _sc_aggregate(x, src, dst, ridx, w):
    mesh = plsc.VectorSubcoreMesh(core_axis_name="c", subcore_axis_name="s")

    @functools.partial(
        pl.kernel,
        mesh=mesh,
        compiler_params=pltpu.CompilerParams(use_tc_tiling_on_sc=False, needs_layout_passes=False),
        out_type=jax.ShapeDtypeStruct((2 * HALF, D), jnp.float32),
        scratch_types=[
            pltpu.VMEM((CH,), jnp.int32),
            pltpu.VMEM((CH,), jnp.int32),
            pltpu.VMEM((CH,), jnp.int32),
            pltpu.VMEM((CH,), jnp.float32),
            pltpu.VMEM((CH, D), jnp.float32),
            pltpu.VMEM((8, D), jnp.float32),
            pltpu.VMEM_SHARED((HALF, D), jnp.float32),
            pltpu.SemaphoreType.DMA,
            pltpu.SemaphoreType.DMA,
            pltpu.SemaphoreType.DMA,
            pltpu.SemaphoreType.DMA,
        ],
    )
    def k(x_hbm, src_hbm, dst_hbm, ridx_hbm, w_hbm, out_hbm,
          gidx, sidx, dbuf, wchunk, rows_v, zbuf, acc,
          sem_a, sem_b, sem_c, sem_d):
        core = lax.axis_index("c")
        sid = lax.axis_index("s")
        rows_per = HALF // NS

        z16 = jnp.zeros((16,), jnp.float32)
        for i in range(8):
            for j in range(D // 16):
                zbuf[i, pl.ds(j * 16, 16)] = z16

        def zacc(t, _):
            pltpu.sync_copy(zbuf, acc.at[pl.ds(sid * rows_per + t * 8, 8), :])
            return 0

        lax.fori_loop(0, rows_per // 8, zacc, 0)

        ebase = sid * EW3
        plsc.subcore_barrier()

        lo = core * HALF
        eids = [jnp.arange(16, dtype=jnp.int32) + g * 16 for g in range(CH // 16)]

        def chunk(c, _):
            off = ebase + c * CH
            i1 = pltpu.async_copy(src_hbm.at[pl.ds(off, CH)], gidx, sem_a)
            i2 = pltpu.async_copy(ridx_hbm.at[pl.ds(off, CH)], sidx, sem_b)
            i3 = pltpu.async_copy(dst_hbm.at[pl.ds(off, CH)], dbuf, sem_c)
            i4 = pltpu.async_copy(w_hbm.at[pl.ds(off, CH)], wchunk, sem_d)
            i1.wait()
            i2.wait()
            i3.wait()
            i4.wait()
            pltpu.async_copy(x_hbm.at[gidx], rows_v, sem_a).wait()
            for g in range(CH // 16):
                dg = dbuf[pl.ds(g * 16, 16)]
                wg = wchunk[pl.ds(g * 16, 16)]
                in_range = (dg >= lo) & (dg < lo + HALF)
                wm = jnp.where(in_range, wg, 0.0)
                eg = eids[g]

                def scale(jb, _):
                    for u in range(4):
                        j = jb * 4 + u
                        js = jnp.full((16,), j, jnp.int32)
                        cv = plsc.load_gather(rows_v, [eg, js])
                        plsc.store_scatter(rows_v, [eg, js], cv * wm)
                    return 0

                lax.fori_loop(0, D // 4, scale, 0)
            pltpu.sync_copy(rows_v, acc.at[sidx], add=True)
            return 0

        lax.fori_loop(0, EW3 // CH, chunk, 0)
        plsc.subcore_barrier()
        pltpu.sync_copy(
            acc.at[pl.ds(sid * rows_per, rows_per), :],
            out_hbm.at[pl.ds(core * HALF + sid * rows_per, rows_per), :],
        )

    return k(x, src, dst, ridx, w)


def _tc_finalize(rows, s0, s1):
    blk = 1000

    def body(r_ref, s0_ref, s1_ref, o_ref):
        inv = 1.0 / (s0_ref[...] + s1_ref[...] + 1e-9)
        o_ref[...] = r_ref[...] * inv

    return pl.pallas_call(
        body,
        grid=(N // blk,),
        in_specs=[
            pl.BlockSpec((blk, D), lambda i: (i, 0)),
            pl.BlockSpec((blk, 1), lambda i: (i, 0)),
            pl.BlockSpec((blk, 1), lambda i: (i, 0)),
        ],
        out_specs=pl.BlockSpec((blk, D), lambda i: (i, 0)),
        out_shape=jax.ShapeDtypeStruct((N, D), jnp.float32),
    )(rows, s0, s1)


def kernel(x, edge_index, npid, W_o, b_o, W_u, b_u, att_w, att_b, pref):
    src = edge_index[0].astype(jnp.int32)
    dst = edge_index[1].astype(jnp.int32)
    npid_i = npid.astype(jnp.int32)

    pad = EP - E
    fill = (jnp.arange(pad, dtype=jnp.int32) % N)
    src_p = jnp.concatenate([src, fill])
    npid_p = jnp.concatenate([npid_i, fill])
    dst_p = jnp.concatenate([dst, jnp.full((pad,), PAD_DST, jnp.int32)])

    hs, p = _tc_dense(x, W_o, b_o, W_u, b_u, pref)

    attw = att_w[:, 0]
    attb = jnp.broadcast_to(att_b, (16,))
    w, spart = _sc_logits(hs, p, src_p, npid_p, dst_p, attw, attb)

    ridx = jnp.minimum(
        jnp.where(dst_p < HALF, dst_p, dst_p - HALF), HALF - 1
    ).astype(jnp.int32)
    rows = _sc_aggregate(x, src_p, dst_p, ridx, w)

    s0 = spart[:N, None]
    s1 = spart[SPAD:SPAD + N, None]
    return _tc_finalize(rows[:N], s0, s1)


# speedup vs baseline: 6.3534x; 6.3534x over previous
import functools

import jax
import jax.numpy as jnp
from jax import lax
from jax.experimental import pallas as pl
from jax.experimental.pallas import tpu as pltpu, tpu_sc as plsc

N = 10000
E = 160000
D = 256
NC = 2
NS = 16
NW = NC * NS
EP = 163840
CH = 128
EW = EP // NW
EW3 = EP // NS
SEC = 2560
HALF = 5120
SPAD = 10752
PAD_DST = 10240


def _tc_dense(x, W_o, b_o, W_u, b_u, pref):
    blk = 1000

    def body(x_ref, wo_ref, bo_ref, wu_ref, bu_ref, pref_ref, hs_ref, p_ref):
        hs_ref[...] = (
            jnp.dot(x_ref[...], wo_ref[...], preferred_element_type=jnp.float32)
            + bo_ref[...]
        )
        p_ref[...] = (
            jnp.dot(pref_ref[...], wu_ref[...], preferred_element_type=jnp.float32)
            + bu_ref[...]
        )

    return pl.pallas_call(
        body,
        grid=(N // blk,),
        in_specs=[
            pl.BlockSpec((blk, D), lambda i: (i, 0)),
            pl.BlockSpec((D, D), lambda i: (0, 0)),
            pl.BlockSpec((1, D), lambda i: (0, 0)),
            pl.BlockSpec((D, D), lambda i: (0, 0)),
            pl.BlockSpec((1, D), lambda i: (0, 0)),
            pl.BlockSpec((blk, D), lambda i: (i, 0)),
        ],
        out_specs=[
            pl.BlockSpec((blk, D), lambda i: (i, 0)),
            pl.BlockSpec((blk, D), lambda i: (i, 0)),
        ],
        out_shape=[
            jax.ShapeDtypeStruct((N, D), jnp.float32),
            jax.ShapeDtypeStruct((N, D), jnp.float32),
        ],
    )(x, W_o, b_o[None, :], W_u, b_u[None, :], pref)


def _sc_logits(hs, p, src, npid, dst, attw, attb):
    mesh = plsc.VectorSubcoreMesh(core_axis_name="c", subcore_axis_name="s")

    @functools.partial(
        pl.kernel,
        mesh=mesh,
        compiler_params=pltpu.CompilerParams(use_tc_tiling_on_sc=False, needs_layout_passes=False),
        out_type=[
            jax.ShapeDtypeStruct((EP,), jnp.float32),
            jax.ShapeDtypeStruct((NC * SPAD,), jnp.float32),
        ],
        scratch_types=[
            pltpu.VMEM((CH,), jnp.int32),
            pltpu.VMEM((CH,), jnp.int32),
            pltpu.VMEM((CH,), jnp.int32),
            pltpu.VMEM((CH, D), jnp.float32),
            pltpu.VMEM((CH, D), jnp.float32),
            pltpu.VMEM((EW,), jnp.float32),
            pltpu.VMEM((D,), jnp.float32),
            pltpu.VMEM((16,), jnp.float32),
            pltpu.VMEM((SPAD // NS,), jnp.float32),
            pltpu.VMEM_SHARED((SPAD,), jnp.float32),
            pltpu.SemaphoreType.DMA,
            pltpu.SemaphoreType.DMA,
            pltpu.SemaphoreType.DMA,
        ],
    )
    def k(hs_hbm, p_hbm, src_hbm, npid_hbm, dst_hbm, attw_hbm, attb_hbm,
          w_out, spart_out, gx, gp, didx,
          rows_a, rows_b, w_all, attw_v, attb_v, zbuf, s_acc,
          sem_a, sem_b, sem_c):
        core = lax.axis_index("c")
        sid = lax.axis_index("s")
        wid = sid * NC + core
        zslice = SPAD // NS

        z16 = jnp.zeros((16,), jnp.float32)

        def zfill(i, _):
            zbuf[pl.ds(i * 16, 16)] = z16
            return 0

        lax.fori_loop(0, zslice // 16, zfill, 0)
        pltpu.sync_copy(zbuf, s_acc.at[pl.ds(sid * zslice, zslice)])

        ebase = wid * EW
        pltpu.sync_copy(attw_hbm, attw_v)
        pltpu.sync_copy(attb_hbm, attb_v)
        plsc.subcore_barrier()

        attb0 = attb_v[...]
        lane16 = jnp.arange(16, dtype=jnp.int32)
        z16f = jnp.zeros((16,), jnp.float32)

        def chunk(c, _):
            coff = c * CH
            off = ebase + coff
            i1 = pltpu.async_copy(src_hbm.at[pl.ds(off, CH)], gx, sem_a)
            i2 = pltpu.async_copy(npid_hbm.at[pl.ds(off, CH)], gp, sem_b)
            i3 = pltpu.async_copy(dst_hbm.at[pl.ds(off, CH)], didx, sem_c)
            i1.wait()
            i2.wait()
            i3.wait()
            ca = pltpu.async_copy(hs_hbm.at[gx], rows_a, sem_a)
            cb = pltpu.async_copy(p_hbm.at[gp], rows_b, sem_b)
            ca.wait()
            cb.wait()
            aws = [attw_v[pl.ds(fb * 16, 16)] for fb in range(D // 16)]
            for g in range(CH // 16):

                def edge(u, wvec):
                    e = g * 16 + u
                    facc = z16f
                    for fb in range(D // 16):
                        hv = rows_a[e, pl.ds(fb * 16, 16)]
                        pv = rows_b[e, pl.ds(fb * 16, 16)]
                        h = hv + pv
                        lr = jnp.where(h > 0.0, h, h * 0.01)
                        facc = facc + lr * aws[fb]
                    a = jnp.sum(facc)
                    return jnp.where(lane16 == u, a, wvec)

                wvec = lax.fori_loop(0, 16, edge, z16f)
                w_all[pl.ds(coff + g * 16, 16)] = jnp.exp(wvec + attb0)
            pltpu.sync_copy(w_all.at[pl.ds(coff, CH)],
                            s_acc.at[didx], add=True)
            return 0

        lax.fori_loop(0, EW // CH, chunk, 0)

        pltpu.sync_copy(w_all, w_out.at[pl.ds(ebase, EW)])
        plsc.subcore_barrier()
        pltpu.sync_copy(s_acc.at[pl.ds(sid * zslice, zslice)],
                        spart_out.at[pl.ds(core * SPAD + sid * zslice, zslice)])

    return k(hs, p, src, npid, dst, attw, attb)


def _sc_aggregate(x, src, dst, ridx, w):
    mesh = plsc.VectorSubcoreMesh(core_axis_name="c", subcore_axis_name="s")

    @functools.partial(
        pl.kernel,
        mesh=mesh,
        compiler_params=pltpu.CompilerParams(use_tc_tiling_on_sc=False, needs_layout_passes=False),
        out_type=jax.ShapeDtypeStruct((2 * HALF, D), jnp.float32),
        scratch_types=[
            pltpu.VMEM((CH,), jnp.int32),
            pltpu.VMEM((CH,), jnp.int32),
            pltpu.VMEM((CH,), jnp.int32),
            pltpu.VMEM((CH,), jnp.float32),
            pltpu.VMEM((CH, D), jnp.float32),
            pltpu.VMEM((8, D), jnp.float32),
            pltpu.VMEM_SHARED((HALF, D), jnp.float32),
            pltpu.SemaphoreType.DMA,
            pltpu.SemaphoreType.DMA,
            pltpu.SemaphoreType.DMA,
            pltpu.SemaphoreType.DMA,
        ],
    )
    def k(x_hbm, src_hbm, dst_hbm, ridx_hbm, w_hbm, out_hbm,
          gidx, sidx, dbuf, wchunk, rows_v, zbuf, acc,
          sem_a, sem_b, sem_c, sem_d):
        core = lax.axis_index("c")
        sid = lax.axis_index("s")
        rows_per = HALF // NS

        z16 = jnp.zeros((16,), jnp.float32)
        for i in range(8):
            for j in range(D // 16):
                zbuf[i, pl.ds(j * 16, 16)] = z16

        def zacc(t, _):
            pltpu.sync_copy(zbuf, acc.at[pl.ds(sid * rows_per + t * 8, 8), :])
            return 0

        lax.fori_loop(0, rows_per // 8, zacc, 0)

        ebase = sid * EW3
        plsc.subcore_barrier()

        lo = core * HALF
        lane16 = jnp.arange(16, dtype=jnp.int32)

        def chunk(c, _):
            off = ebase + c * CH
            i1 = pltpu.async_copy(src_hbm.at[pl.ds(off, CH)], gidx, sem_a)
            i2 = pltpu.async_copy(ridx_hbm.at[pl.ds(off, CH)], sidx, sem_b)
            i3 = pltpu.async_copy(dst_hbm.at[pl.ds(off, CH)], dbuf, sem_c)
            i4 = pltpu.async_copy(w_hbm.at[pl.ds(off, CH)], wchunk, sem_d)
            i1.wait()
            i2.wait()
            i3.wait()
            i4.wait()
            pltpu.async_copy(x_hbm.at[gidx], rows_v, sem_a).wait()
            for g in range(CH // 16):
                dg = dbuf[pl.ds(g * 16, 16)]
                wg = wchunk[pl.ds(g * 16, 16)]
                in_range = (dg >= lo) & (dg < lo + HALF)
                wm = jnp.where(in_range, wg, 0.0)

                def scale(u, _):
                    e = g * 16 + u
                    ws = jnp.sum(jnp.where(lane16 == u, wm, 0.0))
                    wb = jnp.full((16,), ws, jnp.float32)
                    for fb in range(D // 16):
                        r = rows_v[e, pl.ds(fb * 16, 16)]
                        rows_v[e, pl.ds(fb * 16, 16)] = r * wb
                    return 0

                lax.fori_loop(0, 16, scale, 0)
            pltpu.sync_copy(rows_v, acc.at[sidx], add=True)
            return 0

        lax.fori_loop(0, EW3 // CH, chunk, 0)
        plsc.subcore_barrier()
        pltpu.sync_copy(
            acc.at[pl.ds(sid * rows_per, rows_per), :],
            out_hbm.at[pl.ds(core * HALF + sid * rows_per, rows_per), :],
        )

    return k(x, src, dst, ridx, w)


def _tc_finalize(rows, s0, s1):
    blk = 1000

    def body(r_ref, s0_ref, s1_ref, o_ref):
        inv = 1.0 / (s0_ref[...] + s1_ref[...] + 1e-9)
        o_ref[...] = r_ref[...] * inv

    return pl.pallas_call(
        body,
        grid=(N // blk,),
        in_specs=[
            pl.BlockSpec((blk, D), lambda i: (i, 0)),
            pl.BlockSpec((blk, 1), lambda i: (i, 0)),
            pl.BlockSpec((blk, 1), lambda i: (i, 0)),
        ],
        out_specs=pl.BlockSpec((blk, D), lambda i: (i, 0)),
        out_shape=jax.ShapeDtypeStruct((N, D), jnp.float32),
    )(rows, s0, s1)


def kernel(x, edge_index, npid, W_o, b_o, W_u, b_u, att_w, att_b, pref):
    src = edge_index[0].astype(jnp.int32)
    dst = edge_index[1].astype(jnp.int32)
    npid_i = npid.astype(jnp.int32)

    pad = EP - E
    fill = (jnp.arange(pad, dtype=jnp.int32) % N)
    src_p = jnp.concatenate([src, fill])
    npid_p = jnp.concatenate([npid_i, fill])
    dst_p = jnp.concatenate([dst, jnp.full((pad,), PAD_DST, jnp.int32)])

    hs, p = _tc_dense(x, W_o, b_o, W_u, b_u, pref)

    attw = att_w[:, 0]
    attb = jnp.broadcast_to(att_b, (16,))
    w, spart = _sc_logits(hs, p, src_p, npid_p, dst_p, attw, attb)

    ridx = jnp.minimum(
        jnp.where(dst_p < HALF, dst_p, dst_p - HALF), HALF - 1
    ).astype(jnp.int32)
    rows = _sc_aggregate(x, src_p, dst_p, ridx, w)

    s0 = spart[:N, None]
    s1 = spart[SPAD:SPAD + N, None]
    return _tc_finalize(rows[:N], s0, s1)


# speedup vs baseline: 7.4404x; 1.1711x over previous
import functools

import jax
import jax.numpy as jnp
from jax import lax
from jax.experimental import pallas as pl
from jax.experimental.pallas import tpu as pltpu, tpu_sc as plsc

N = 10000
E = 160000
D = 256
NC = 2
NS = 16
NW = NC * NS
EP = 163840
CH = 128
EW = EP // NW
EW3 = EP // NS
CHA = 64
HALF = 5120
SPAD = 10752
PAD_DST = 10240


def _tc_dense(x, W_o, b_o, W_u, b_u, pref):
    blk = 1000

    def body(x_ref, wo_ref, bo_ref, wu_ref, bu_ref, pref_ref, hs_ref, p_ref):
        hs_ref[...] = (
            jnp.dot(x_ref[...], wo_ref[...], preferred_element_type=jnp.float32)
            + bo_ref[...]
        )
        p_ref[...] = (
            jnp.dot(pref_ref[...], wu_ref[...], preferred_element_type=jnp.float32)
            + bu_ref[...]
        )

    return pl.pallas_call(
        body,
        grid=(N // blk,),
        in_specs=[
            pl.BlockSpec((blk, D), lambda i: (i, 0)),
            pl.BlockSpec((D, D), lambda i: (0, 0)),
            pl.BlockSpec((1, D), lambda i: (0, 0)),
            pl.BlockSpec((D, D), lambda i: (0, 0)),
            pl.BlockSpec((1, D), lambda i: (0, 0)),
            pl.BlockSpec((blk, D), lambda i: (i, 0)),
        ],
        out_specs=[
            pl.BlockSpec((blk, D), lambda i: (i, 0)),
            pl.BlockSpec((blk, D), lambda i: (i, 0)),
        ],
        out_shape=[
            jax.ShapeDtypeStruct((N, D), jnp.float32),
            jax.ShapeDtypeStruct((N, D), jnp.float32),
        ],
    )(x, W_o, b_o[None, :], W_u, b_u[None, :], pref)


def _sc_logits(hs, p, src, npid, dst, attw, attb):
    mesh = plsc.VectorSubcoreMesh(core_axis_name="c", subcore_axis_name="s")

    @functools.partial(
        pl.kernel,
        mesh=mesh,
        compiler_params=pltpu.CompilerParams(use_tc_tiling_on_sc=False, needs_layout_passes=False),
        out_type=[
            jax.ShapeDtypeStruct((EP,), jnp.float32),
            jax.ShapeDtypeStruct((NC * SPAD,), jnp.float32),
        ],
        scratch_types=[
            pltpu.VMEM((CH,), jnp.int32),
            pltpu.VMEM((CH,), jnp.int32),
            pltpu.VMEM((CH,), jnp.int32),
            pltpu.VMEM((CH, D), jnp.float32),
            pltpu.VMEM((CH, D), jnp.float32),
            pltpu.VMEM((EW,), jnp.float32),
            pltpu.VMEM((D,), jnp.float32),
            pltpu.VMEM((16,), jnp.float32),
            pltpu.VMEM((SPAD // NS,), jnp.float32),
            pltpu.VMEM_SHARED((SPAD,), jnp.float32),
            pltpu.SemaphoreType.DMA,
            pltpu.SemaphoreType.DMA,
            pltpu.SemaphoreType.DMA,
        ],
    )
    def k(hs_hbm, p_hbm, src_hbm, npid_hbm, dst_hbm, attw_hbm, attb_hbm,
          w_out, spart_out, gx, gp, didx,
          rows_a, rows_b, w_all, attw_v, attb_v, zbuf, s_acc,
          sem_a, sem_b, sem_c):
        core = lax.axis_index("c")
        sid = lax.axis_index("s")
        wid = sid * NC + core
        zslice = SPAD // NS

        z16 = jnp.zeros((16,), jnp.float32)

        def zfill(i, _):
            zbuf[pl.ds(i * 16, 16)] = z16
            return 0

        lax.fori_loop(0, zslice // 16, zfill, 0)
        pltpu.sync_copy(zbuf, s_acc.at[pl.ds(sid * zslice, zslice)])

        ebase = wid * EW
        pltpu.sync_copy(attw_hbm, attw_v)
        pltpu.sync_copy(attb_hbm, attb_v)
        plsc.subcore_barrier()

        attb0 = attb_v[...]
        lane16 = jnp.arange(16, dtype=jnp.int32)
        z16f = jnp.zeros((16,), jnp.float32)

        def chunk(c, _):
            coff = c * CH
            off = ebase + coff
            i1 = pltpu.async_copy(src_hbm.at[pl.ds(off, CH)], gx, sem_a)
            i2 = pltpu.async_copy(npid_hbm.at[pl.ds(off, CH)], gp, sem_b)
            i3 = pltpu.async_copy(dst_hbm.at[pl.ds(off, CH)], didx, sem_c)
            i1.wait()
            i2.wait()
            i3.wait()
            ca = pltpu.async_copy(hs_hbm.at[gx], rows_a, sem_a)
            cb = pltpu.async_copy(p_hbm.at[gp], rows_b, sem_b)
            ca.wait()
            cb.wait()
            aws = [attw_v[pl.ds(fb * 16, 16)] for fb in range(D // 16)]
            for g in range(CH // 16):

                def edge(u, wvec):
                    e = g * 16 + u
                    facc = z16f
                    for fb in range(D // 16):
                        hv = rows_a[e, pl.ds(fb * 16, 16)]
                        pv = rows_b[e, pl.ds(fb * 16, 16)]
                        h = hv + pv
                        lr = jnp.where(h > 0.0, h, h * 0.01)
                        facc = facc + lr * aws[fb]
                    a = jnp.sum(facc)
                    return jnp.where(lane16 == u, a, wvec)

                wvec = lax.fori_loop(0, 16, edge, z16f)
                w_all[pl.ds(coff + g * 16, 16)] = jnp.exp(wvec + attb0)
            pltpu.sync_copy(w_all.at[pl.ds(coff, CH)],
                            s_acc.at[didx], add=True)
            return 0

        lax.fori_loop(0, EW // CH, chunk, 0)

        pltpu.sync_copy(w_all, w_out.at[pl.ds(ebase, EW)])
        plsc.subcore_barrier()
        pltpu.sync_copy(s_acc.at[pl.ds(sid * zslice, zslice)],
                        spart_out.at[pl.ds(core * SPAD + sid * zslice, zslice)])

    return k(hs, p, src, npid, dst, attw, attb)


def _sc_aggregate(x, src, dst, ridx, w):
    mesh = plsc.VectorSubcoreMesh(core_axis_name="c", subcore_axis_name="s")

    @functools.partial(
        pl.kernel,
        mesh=mesh,
        compiler_params=pltpu.CompilerParams(use_tc_tiling_on_sc=False, needs_layout_passes=False),
        out_type=jax.ShapeDtypeStruct((2 * HALF, D), jnp.float32),
        scratch_types=[
            pltpu.VMEM((2, CHA), jnp.int32),
            pltpu.VMEM((2, CHA), jnp.int32),
            pltpu.VMEM((2, CHA), jnp.int32),
            pltpu.VMEM((2, CHA), jnp.float32),
            pltpu.VMEM((CHA, D), jnp.float32),
            pltpu.VMEM((CHA, D), jnp.float32),
            pltpu.VMEM((8, D), jnp.float32),
            pltpu.VMEM_SHARED((HALF, D), jnp.float32),
            pltpu.SemaphoreType.DMA,
            pltpu.SemaphoreType.DMA,
            pltpu.SemaphoreType.DMA,
        ],
    )
    def k(x_hbm, src_hbm, dst_hbm, ridx_hbm, w_hbm, out_hbm,
          gidx, sidx, dbuf, wchunk, rows0, rows1, zbuf, acc,
          semi0, semi1, semg):
        core = lax.axis_index("c")
        sid = lax.axis_index("s")
        rows_per = HALF // NS
        rows_b = (rows0, rows1)
        semi = (semi0, semi1)

        z16 = jnp.zeros((16,), jnp.float32)
        for i in range(8):
            for j in range(D // 16):
                zbuf[i, pl.ds(j * 16, 16)] = z16

        def zacc(t, _):
            pltpu.sync_copy(zbuf, acc.at[pl.ds(sid * rows_per + t * 8, 8), :])
            return 0

        lax.fori_loop(0, rows_per // 8, zacc, 0)

        ebase = sid * EW3
        plsc.subcore_barrier()

        lo = core * HALF
        lane16 = jnp.arange(16, dtype=jnp.int32)
        ncha = EW3 // CHA

        def issue_idx(b, c):
            off = ebase + jnp.minimum(c, ncha - 1) * CHA
            pltpu.async_copy(src_hbm.at[pl.ds(off, CHA)], gidx.at[b], semi[b])
            pltpu.async_copy(ridx_hbm.at[pl.ds(off, CHA)], sidx.at[b], semi[b])
            pltpu.async_copy(dst_hbm.at[pl.ds(off, CHA)], dbuf.at[b], semi[b])
            pltpu.async_copy(w_hbm.at[pl.ds(off, CHA)], wchunk.at[b], semi[b])

        def wait_idx(b):
            pltpu.make_async_copy(src_hbm.at[pl.ds(0, CHA)], gidx.at[b],
                                  semi[b]).wait()
            pltpu.make_async_copy(ridx_hbm.at[pl.ds(0, CHA)], sidx.at[b],
                                  semi[b]).wait()
            pltpu.make_async_copy(dst_hbm.at[pl.ds(0, CHA)], dbuf.at[b],
                                  semi[b]).wait()
            pltpu.make_async_copy(w_hbm.at[pl.ds(0, CHA)], wchunk.at[b],
                                  semi[b]).wait()

        def issue_gather(b):
            pltpu.async_copy(x_hbm.at[gidx.at[b]], rows_b[b], semg)

        def wait_gather(b):
            pltpu.make_async_copy(x_hbm.at[pl.ds(0, CHA)], rows_b[b],
                                  semg).wait()

        def compute(b):
            rv = rows_b[b]
            for g in range(CHA // 16):
                dg = dbuf[b, pl.ds(g * 16, 16)]
                wg = wchunk[b, pl.ds(g * 16, 16)]
                in_range = (dg >= lo) & (dg < lo + HALF)
                wm = jnp.where(in_range, wg, 0.0)

                def scale(u, _):
                    e = g * 16 + u
                    ws = jnp.sum(jnp.where(lane16 == u, wm, 0.0))
                    wb = jnp.full((16,), ws, jnp.float32)
                    for fb in range(D // 16):
                        r = rv[e, pl.ds(fb * 16, 16)]
                        rv[e, pl.ds(fb * 16, 16)] = r * wb
                    return 0

                lax.fori_loop(0, 16, scale, 0)
            pltpu.sync_copy(rv, acc.at[sidx.at[b]], add=True)

        issue_idx(0, 0)
        wait_idx(0)
        issue_gather(0)
        issue_idx(1, 1)

        def step(s, _):
            c = s * 2
            wait_gather(0)
            wait_idx(1)
            issue_gather(1)
            compute(0)
            issue_idx(0, c + 2)
            wait_gather(1)
            wait_idx(0)
            issue_gather(0)
            compute(1)
            issue_idx(1, c + 3)
            return 0

        lax.fori_loop(0, ncha // 2, step, 0)
        wait_gather(0)
        wait_idx(1)
        plsc.subcore_barrier()
        pltpu.sync_copy(
            acc.at[pl.ds(sid * rows_per, rows_per), :],
            out_hbm.at[pl.ds(core * HALF + sid * rows_per, rows_per), :],
        )

    return k(x, src, dst, ridx, w)


def _tc_finalize(rows, s0, s1):
    blk = 1000

    def body(r_ref, s0_ref, s1_ref, o_ref):
        inv = 1.0 / (s0_ref[...] + s1_ref[...] + 1e-9)
        o_ref[...] = r_ref[...] * inv

    return pl.pallas_call(
        body,
        grid=(N // blk,),
        in_specs=[
            pl.BlockSpec((blk, D), lambda i: (i, 0)),
            pl.BlockSpec((blk, 1), lambda i: (i, 0)),
            pl.BlockSpec((blk, 1), lambda i: (i, 0)),
        ],
        out_specs=pl.BlockSpec((blk, D), lambda i: (i, 0)),
        out_shape=jax.ShapeDtypeStruct((N, D), jnp.float32),
    )(rows, s0, s1)


def kernel(x, edge_index, npid, W_o, b_o, W_u, b_u, att_w, att_b, pref):
    src = edge_index[0].astype(jnp.int32)
    dst = edge_index[1].astype(jnp.int32)
    npid_i = npid.astype(jnp.int32)

    pad = EP - E
    fill = (jnp.arange(pad, dtype=jnp.int32) % N)
    src_p = jnp.concatenate([src, fill])
    npid_p = jnp.concatenate([npid_i, fill])
    dst_p = jnp.concatenate([dst, jnp.full((pad,), PAD_DST, jnp.int32)])

    hs, p = _tc_dense(x, W_o, b_o, W_u, b_u, pref)

    attw = att_w[:, 0]
    attb = jnp.broadcast_to(att_b, (16,))
    w, spart = _sc_logits(hs, p, src_p, npid_p, dst_p, attw, attb)

    ridx = jnp.minimum(
        jnp.where(dst_p < HALF, dst_p, dst_p - HALF), HALF - 1
    ).astype(jnp.int32)
    rows = _sc_aggregate(x, src_p, dst_p, ridx, w)

    s0 = spart[:N, None]
    s1 = spart[SPAD:SPAD + N, None]
    return _tc_finalize(rows[:N], s0, s1)


# speedup vs baseline: 8.7583x; 1.1771x over previous
import functools

import jax
import jax.numpy as jnp
from jax import lax
from jax.experimental import pallas as pl
from jax.experimental.pallas import tpu as pltpu, tpu_sc as plsc

N = 10000
E = 160000
D = 256
NC = 2
NS = 16
NW = NC * NS
EP = 163840
CH = 128
EW = EP // NW
EW3 = EP // NS
CHA = 64
HALF = 5120
SPAD = 10752
PAD_DST = 10240


def _tc_dense(x, W_o, b_o, W_u, b_u, pref):
    blk = 1000

    def body(x_ref, wo_ref, bo_ref, wu_ref, bu_ref, pref_ref, hs_ref, p_ref):
        hs_ref[...] = (
            jnp.dot(x_ref[...], wo_ref[...], preferred_element_type=jnp.float32)
            + bo_ref[...]
        )
        p_ref[...] = (
            jnp.dot(pref_ref[...], wu_ref[...], preferred_element_type=jnp.float32)
            + bu_ref[...]
        )

    return pl.pallas_call(
        body,
        grid=(N // blk,),
        in_specs=[
            pl.BlockSpec((blk, D), lambda i: (i, 0)),
            pl.BlockSpec((D, D), lambda i: (0, 0)),
            pl.BlockSpec((1, D), lambda i: (0, 0)),
            pl.BlockSpec((D, D), lambda i: (0, 0)),
            pl.BlockSpec((1, D), lambda i: (0, 0)),
            pl.BlockSpec((blk, D), lambda i: (i, 0)),
        ],
        out_specs=[
            pl.BlockSpec((blk, D), lambda i: (i, 0)),
            pl.BlockSpec((blk, D), lambda i: (i, 0)),
        ],
        out_shape=[
            jax.ShapeDtypeStruct((N, D), jnp.float32),
            jax.ShapeDtypeStruct((N, D), jnp.float32),
        ],
    )(x, W_o, b_o[None, :], W_u, b_u[None, :], pref)


def _sc_logits(hs, p, src, npid, dst, attw, attb):
    mesh = plsc.VectorSubcoreMesh(core_axis_name="c", subcore_axis_name="s")

    @functools.partial(
        pl.kernel,
        mesh=mesh,
        compiler_params=pltpu.CompilerParams(use_tc_tiling_on_sc=False, needs_layout_passes=False),
        out_type=[
            jax.ShapeDtypeStruct((EP,), jnp.float32),
            jax.ShapeDtypeStruct((NC * SPAD,), jnp.float32),
        ],
        scratch_types=[
            pltpu.VMEM((2, CHA), jnp.int32),
            pltpu.VMEM((2, CHA), jnp.int32),
            pltpu.VMEM((2, CHA), jnp.int32),
            pltpu.VMEM((CHA, D), jnp.float32),
            pltpu.VMEM((CHA, D), jnp.float32),
            pltpu.VMEM((CHA, D), jnp.float32),
            pltpu.VMEM((CHA, D), jnp.float32),
            pltpu.VMEM((EW,), jnp.float32),
            pltpu.VMEM((D,), jnp.float32),
            pltpu.VMEM((16,), jnp.float32),
            pltpu.VMEM((SPAD // NS,), jnp.float32),
            pltpu.VMEM_SHARED((SPAD,), jnp.float32),
            pltpu.SemaphoreType.DMA,
            pltpu.SemaphoreType.DMA,
            pltpu.SemaphoreType.DMA,
            pltpu.SemaphoreType.DMA,
        ],
    )
    def k(hs_hbm, p_hbm, src_hbm, npid_hbm, dst_hbm, attw_hbm, attb_hbm,
          w_out, spart_out, gx, gp, didx,
          rows_a0, rows_a1, rows_b0, rows_b1, w_all, attw_v, attb_v,
          zbuf, s_acc, semi0, semi1, semg0, semg1):
        core = lax.axis_index("c")
        sid = lax.axis_index("s")
        wid = sid * NC + core
        zslice = SPAD // NS
        ra = (rows_a0, rows_a1)
        rb = (rows_b0, rows_b1)
        semi = (semi0, semi1)
        semg = (semg0, semg1)

        z16 = jnp.zeros((16,), jnp.float32)

        def zfill(i, _):
            zbuf[pl.ds(i * 16, 16)] = z16
            return 0

        lax.fori_loop(0, zslice // 16, zfill, 0)
        pltpu.sync_copy(zbuf, s_acc.at[pl.ds(sid * zslice, zslice)])

        ebase = wid * EW
        pltpu.sync_copy(attw_hbm, attw_v)
        pltpu.sync_copy(attb_hbm, attb_v)
        plsc.subcore_barrier()

        attb0 = attb_v[...]
        lane16 = jnp.arange(16, dtype=jnp.int32)
        z16f = jnp.zeros((16,), jnp.float32)
        aws = [attw_v[pl.ds(fb * 16, 16)] for fb in range(D // 16)]
        ncha = EW // CHA

        def issue_idx(b, c):
            off = ebase + jnp.minimum(c, ncha - 1) * CHA
            pltpu.async_copy(src_hbm.at[pl.ds(off, CHA)], gx.at[b], semi[b])
            pltpu.async_copy(npid_hbm.at[pl.ds(off, CHA)], gp.at[b], semi[b])
            pltpu.async_copy(dst_hbm.at[pl.ds(off, CHA)], didx.at[b], semi[b])

        def wait_idx(b):
            pltpu.make_async_copy(src_hbm.at[pl.ds(0, CHA)], gx.at[b],
                                  semi[b]).wait()
            pltpu.make_async_copy(npid_hbm.at[pl.ds(0, CHA)], gp.at[b],
                                  semi[b]).wait()
            pltpu.make_async_copy(dst_hbm.at[pl.ds(0, CHA)], didx.at[b],
                                  semi[b]).wait()

        def issue_gather(b):
            pltpu.async_copy(hs_hbm.at[gx.at[b]], ra[b], semg[b])
            pltpu.async_copy(p_hbm.at[gp.at[b]], rb[b], semg[b])

        def wait_gather(b):
            pltpu.make_async_copy(hs_hbm.at[pl.ds(0, CHA)], ra[b],
                                  semg[b]).wait()
            pltpu.make_async_copy(p_hbm.at[pl.ds(0, CHA)], rb[b],
                                  semg[b]).wait()

        def compute(b, c):
            coff = c * CHA
            rav, rbv = ra[b], rb[b]
            for g in range(CHA // 16):

                def edge(u, wvec):
                    e = g * 16 + u
                    facc = z16f
                    for fb in range(D // 16):
                        hv = rav[e, pl.ds(fb * 16, 16)]
                        pv = rbv[e, pl.ds(fb * 16, 16)]
                        h = hv + pv
                        lr = jnp.where(h > 0.0, h, h * 0.01)
                        facc = facc + lr * aws[fb]
                    a = jnp.sum(facc)
                    return jnp.where(lane16 == u, a, wvec)

                wvec = lax.fori_loop(0, 16, edge, z16f)
                w_all[pl.ds(coff + g * 16, 16)] = jnp.exp(wvec + attb0)
            pltpu.sync_copy(w_all.at[pl.ds(coff, CHA)],
                            s_acc.at[didx.at[b]], add=True)

        issue_idx(0, 0)
        wait_idx(0)
        issue_gather(0)
        issue_idx(1, 1)

        def step(s, _):
            c = s * 2
            wait_gather(0)
            wait_idx(1)
            issue_gather(1)
            compute(0, c)
            issue_idx(0, c + 2)
            wait_gather(1)
            wait_idx(0)
            issue_gather(0)
            compute(1, c + 1)
            issue_idx(1, c + 3)
            return 0

        lax.fori_loop(0, ncha // 2, step, 0)
        wait_gather(0)
        wait_idx(1)

        pltpu.sync_copy(w_all, w_out.at[pl.ds(ebase, EW)])
        plsc.subcore_barrier()
        pltpu.sync_copy(s_acc.at[pl.ds(sid * zslice, zslice)],
                        spart_out.at[pl.ds(core * SPAD + sid * zslice, zslice)])

    return k(hs, p, src, npid, dst, attw, attb)


def _sc_aggregate(x, src, dst, ridx, w):
    mesh = plsc.VectorSubcoreMesh(core_axis_name="c", subcore_axis_name="s")

    @functools.partial(
        pl.kernel,
        mesh=mesh,
        compiler_params=pltpu.CompilerParams(use_tc_tiling_on_sc=False, needs_layout_passes=False),
        out_type=jax.ShapeDtypeStruct((2 * HALF, D), jnp.float32),
        scratch_types=[
            pltpu.VMEM((2, CHA), jnp.int32),
            pltpu.VMEM((2, CHA), jnp.int32),
            pltpu.VMEM((2, CHA), jnp.int32),
            pltpu.VMEM((2, CHA), jnp.float32),
            pltpu.VMEM((CHA, D), jnp.float32),
            pltpu.VMEM((CHA, D), jnp.float32),
            pltpu.VMEM((8, D), jnp.float32),
            pltpu.VMEM_SHARED((HALF, D), jnp.float32),
            pltpu.SemaphoreType.DMA,
            pltpu.SemaphoreType.DMA,
            pltpu.SemaphoreType.DMA,
        ],
    )
    def k(x_hbm, src_hbm, dst_hbm, ridx_hbm, w_hbm, out_hbm,
          gidx, sidx, dbuf, wchunk, rows0, rows1, zbuf, acc,
          semi0, semi1, semg):
        core = lax.axis_index("c")
        sid = lax.axis_index("s")
        rows_per = HALF // NS
        rows_b = (rows0, rows1)
        semi = (semi0, semi1)

        z16 = jnp.zeros((16,), jnp.float32)
        for i in range(8):
            for j in range(D // 16):
                zbuf[i, pl.ds(j * 16, 16)] = z16

        def zacc(t, _):
            pltpu.sync_copy(zbuf, acc.at[pl.ds(sid * rows_per + t * 8, 8), :])
            return 0

        lax.fori_loop(0, rows_per // 8, zacc, 0)

        ebase = sid * EW3
        plsc.subcore_barrier()

        lo = core * HALF
        lane16 = jnp.arange(16, dtype=jnp.int32)
        ncha = EW3 // CHA

        def issue_idx(b, c):
            off = ebase + jnp.minimum(c, ncha - 1) * CHA
            pltpu.async_copy(src_hbm.at[pl.ds(off, CHA)], gidx.at[b], semi[b])
            pltpu.async_copy(ridx_hbm.at[pl.ds(off, CHA)], sidx.at[b], semi[b])
            pltpu.async_copy(dst_hbm.at[pl.ds(off, CHA)], dbuf.at[b], semi[b])
            pltpu.async_copy(w_hbm.at[pl.ds(off, CHA)], wchunk.at[b], semi[b])

        def wait_idx(b):
            pltpu.make_async_copy(src_hbm.at[pl.ds(0, CHA)], gidx.at[b],
                                  semi[b]).wait()
            pltpu.make_async_copy(ridx_hbm.at[pl.ds(0, CHA)], sidx.at[b],
                                  semi[b]).wait()
            pltpu.make_async_copy(dst_hbm.at[pl.ds(0, CHA)], dbuf.at[b],
                                  semi[b]).wait()
            pltpu.make_async_copy(w_hbm.at[pl.ds(0, CHA)], wchunk.at[b],
                                  semi[b]).wait()

        def issue_gather(b):
            pltpu.async_copy(x_hbm.at[gidx.at[b]], rows_b[b], semg)

        def wait_gather(b):
            pltpu.make_async_copy(x_hbm.at[pl.ds(0, CHA)], rows_b[b],
                                  semg).wait()

        def compute(b):
            rv = rows_b[b]
            for g in range(CHA // 16):
                dg = dbuf[b, pl.ds(g * 16, 16)]
                wg = wchunk[b, pl.ds(g * 16, 16)]
                in_range = (dg >= lo) & (dg < lo + HALF)
                wm = jnp.where(in_range, wg, 0.0)

                def scale(u, _):
                    e = g * 16 + u
                    ws = jnp.sum(jnp.where(lane16 == u, wm, 0.0))
                    wb = jnp.full((16,), ws, jnp.float32)
                    for fb in range(D // 16):
                        r = rv[e, pl.ds(fb * 16, 16)]
                        rv[e, pl.ds(fb * 16, 16)] = r * wb
                    return 0

                lax.fori_loop(0, 16, scale, 0)
            pltpu.sync_copy(rv, acc.at[sidx.at[b]], add=True)

        issue_idx(0, 0)
        wait_idx(0)
        issue_gather(0)
        issue_idx(1, 1)

        def step(s, _):
            c = s * 2
            wait_gather(0)
            wait_idx(1)
            issue_gather(1)
            compute(0)
            issue_idx(0, c + 2)
            wait_gather(1)
            wait_idx(0)
            issue_gather(0)
            compute(1)
            issue_idx(1, c + 3)
            return 0

        lax.fori_loop(0, ncha // 2, step, 0)
        wait_gather(0)
        wait_idx(1)
        plsc.subcore_barrier()
        pltpu.sync_copy(
            acc.at[pl.ds(sid * rows_per, rows_per), :],
            out_hbm.at[pl.ds(core * HALF + sid * rows_per, rows_per), :],
        )

    return k(x, src, dst, ridx, w)


def _tc_finalize(rows, s0, s1):
    blk = 1000

    def body(r_ref, s0_ref, s1_ref, o_ref):
        inv = 1.0 / (s0_ref[...] + s1_ref[...] + 1e-9)
        o_ref[...] = r_ref[...] * inv

    return pl.pallas_call(
        body,
        grid=(N // blk,),
        in_specs=[
            pl.BlockSpec((blk, D), lambda i: (i, 0)),
            pl.BlockSpec((blk, 1), lambda i: (i, 0)),
            pl.BlockSpec((blk, 1), lambda i: (i, 0)),
        ],
        out_specs=pl.BlockSpec((blk, D), lambda i: (i, 0)),
        out_shape=jax.ShapeDtypeStruct((N, D), jnp.float32),
    )(rows, s0, s1)


def kernel(x, edge_index, npid, W_o, b_o, W_u, b_u, att_w, att_b, pref):
    src = edge_index[0].astype(jnp.int32)
    dst = edge_index[1].astype(jnp.int32)
    npid_i = npid.astype(jnp.int32)

    pad = EP - E
    fill = (jnp.arange(pad, dtype=jnp.int32) % N)
    src_p = jnp.concatenate([src, fill])
    npid_p = jnp.concatenate([npid_i, fill])
    dst_p = jnp.concatenate([dst, jnp.full((pad,), PAD_DST, jnp.int32)])

    hs, p = _tc_dense(x, W_o, b_o, W_u, b_u, pref)

    attw = att_w[:, 0]
    attb = jnp.broadcast_to(att_b, (16,))
    w, spart = _sc_logits(hs, p, src_p, npid_p, dst_p, attw, attb)

    ridx = jnp.minimum(
        jnp.where(dst_p < HALF, dst_p, dst_p - HALF), HALF - 1
    ).astype(jnp.int32)
    rows = _sc_aggregate(x, src_p, dst_p, ridx, w)

    s0 = spart[:N, None]
    s1 = spart[SPAD:SPAD + N, None]
    return _tc_finalize(rows[:N], s0, s1)
